# Initial kernel scaffold; baseline (speedup 1.0000x reference)
import jax
import jax.numpy as jnp
from jax.experimental import pallas as pl


def kernel(x, Wr, W1, W2, W3, Ws1, Ws2, Ws3):
    raise NotImplementedError("write your pallas kernel here")


# speedup vs baseline: 1.9813x; 1.9813x over previous
import functools

import jax
import jax.numpy as jnp
from jax import lax
from jax.experimental import pallas as pl
from jax.experimental.pallas import tpu as pltpu

D_MODEL = 768
HIDDEN = 1536
NUM_ROUTED = 7
N_TOK = 2048
BT = 512
NT = N_TOK // BT


def _moe_body(x_ref, wr_ref, w1_ref, w2_ref, w3_ref, ws1_ref, ws2_ref,
              ws3_ref, out_ref):
    e = pl.program_id(0)
    t = pl.program_id(1)

    xb = x_ref[...]

    logits = jnp.dot(xb, wr_ref[...], preferred_element_type=jnp.float32)
    iot = lax.broadcasted_iota(jnp.int32, (BT, NUM_ROUTED), 1)
    m1 = jnp.max(logits, axis=1, keepdims=True)
    i1 = jnp.min(jnp.where(logits == m1, iot, NUM_ROUTED), axis=1,
                 keepdims=True)
    masked = jnp.where(iot == i1, -1e30, logits)
    m2 = jnp.max(masked, axis=1, keepdims=True)
    i2 = jnp.min(jnp.where(masked == m2, iot, NUM_ROUTED), axis=1,
                 keepdims=True)
    e2 = jnp.exp(m2 - m1)
    w_first = 1.0 / (1.0 + e2)
    w_second = 1.0 - w_first

    is_shared = e == NUM_ROUTED
    g = jnp.where(is_shared, 1.0,
                  jnp.where(i1 == e, w_first,
                            jnp.where(i2 == e, w_second, 0.0)))

    w1b = jnp.where(is_shared, ws1_ref[0], w1_ref[0]).astype(jnp.bfloat16)
    w2b = jnp.where(is_shared, ws2_ref[0], w2_ref[0]).astype(jnp.bfloat16)
    w3b = jnp.where(is_shared, ws3_ref[0], w3_ref[0]).astype(jnp.bfloat16)

    xbb = xb.astype(jnp.bfloat16)
    a1 = jnp.dot(xbb, w1b, preferred_element_type=jnp.float32)
    a2 = jnp.dot(xbb, w2b, preferred_element_type=jnp.float32)
    act = a1 * (1.0 / (1.0 + jnp.exp(-a1))) * a2
    actb = (act * g).astype(jnp.bfloat16)
    contrib = jnp.dot(actb, w3b, preferred_element_type=jnp.float32)

    sl = pl.ds(t * BT, BT)

    @pl.when(e == 0)
    def _():
        out_ref[sl, :] = contrib

    @pl.when(e > 0)
    def _():
        out_ref[sl, :] = out_ref[sl, :] + contrib


@jax.jit
def _moe_dense(xf, Wr, W1, W2, W3, Ws1, Ws2, Ws3):
    return pl.pallas_call(
        _moe_body,
        grid=(NUM_ROUTED + 1, NT),
        in_specs=[
            pl.BlockSpec((BT, D_MODEL), lambda e, t: (t, 0)),
            pl.BlockSpec((D_MODEL, NUM_ROUTED), lambda e, t: (0, 0)),
            pl.BlockSpec((1, D_MODEL, HIDDEN),
                         lambda e, t: (jnp.minimum(e, NUM_ROUTED - 1), 0, 0)),
            pl.BlockSpec((1, D_MODEL, HIDDEN),
                         lambda e, t: (jnp.minimum(e, NUM_ROUTED - 1), 0, 0)),
            pl.BlockSpec((1, HIDDEN, D_MODEL),
                         lambda e, t: (jnp.minimum(e, NUM_ROUTED - 1), 0, 0)),
            pl.BlockSpec((1, D_MODEL, HIDDEN), lambda e, t: (0, 0, 0)),
            pl.BlockSpec((1, D_MODEL, HIDDEN), lambda e, t: (0, 0, 0)),
            pl.BlockSpec((1, HIDDEN, D_MODEL), lambda e, t: (0, 0, 0)),
        ],
        out_specs=pl.BlockSpec((N_TOK, D_MODEL), lambda e, t: (0, 0)),
        out_shape=jax.ShapeDtypeStruct((N_TOK, D_MODEL), jnp.float32),
    )(xf, Wr, W1, W2, W3, Ws1, Ws2, Ws3)


def kernel(x, Wr, W1, W2, W3, Ws1, Ws2, Ws3):
    orig_shape = x.shape
    xf = x.reshape(-1, orig_shape[-1])
    out = _moe_dense(xf, Wr, W1, W2, W3, Ws1, Ws2, Ws3)
    return out.reshape(orig_shape)
